# baseline (speedup 1.0000x reference)
import functools

import jax
import jax.numpy as jnp
from jax import lax
from jax.experimental import pallas as pl
from jax.experimental.pallas import tpu as pltpu
from jax.experimental.pallas import tpu_sc as plsc

F = 26
CARD = 100001
D = 32
B = 16384
NC = 2
NS = 16
NW = NC * NS
XH = 8192
XC = 4096
NR = B // XC
L = 16


def _sc_body(xt, wt, ot, tbl, xh0, xh1, oh0, oh1, tsem, xs0, xs1, os0, os1):
    e = lax.axis_index("s") * NC + lax.axis_index("c")
    xh = [xh0, xh1]
    oh = [oh0, oh1]
    xsem = [xs0, xs1]
    osem = [os0, os1]

    def issue_x(f, h):
        pltpu.async_copy(xt.at[f, pl.ds(h * XH, XH)], xh[h], xsem[h])

    def drain_x(h):
        pltpu.make_async_copy(xt.at[0, pl.ds(0, XH)], xh[h], xsem[h]).wait()

    def drain_o(s):
        pltpu.make_async_copy(oh[s], ot.at[0, 0, pl.ds(0, XC)], osem[s]).wait()

    issue_x(0, 0)

    def do_field(f, carry):
        tcp = pltpu.async_copy(wt.at[f, e], tbl, tsem)
        issue_x(f, 1)

        @pl.when(f > 0)
        def _():
            drain_o(0)
            drain_o(1)

        tcp.wait()
        for r in range(NR):
            s = r % 2
            h = r // 2
            if r % 2 == 0:
                drain_x(h)
            if r >= 2:
                drain_o(s)

            def grp(i, carry2):
                for u in range(8):
                    p = (i * 8 + u) * L
                    idx = xh[h][pl.ds((r % 2) * XC + p, L)]
                    oh[s][pl.ds(p, L)] = plsc.load_gather(tbl, [idx])
                return carry2
            lax.fori_loop(0, XC // L // 8, grp, 0)

            if r == 1:
                @pl.when(f < F - 1)
                def _():
                    issue_x(f + 1, 0)

            pltpu.async_copy(oh[s], ot.at[f, e, pl.ds(r * XC, XC)], osem[s])
        return carry

    lax.fori_loop(0, F, do_field, 0)
    drain_o(0)
    drain_o(1)


_emb = functools.partial(
    pl.kernel,
    mesh=plsc.VectorSubcoreMesh(core_axis_name="c", subcore_axis_name="s"),
    out_type=jax.ShapeDtypeStruct((F, D, B), jnp.float32),
    compiler_params=pltpu.CompilerParams(needs_layout_passes=False),
    scratch_types=[
        pltpu.VMEM((CARD,), jnp.float32),
        pltpu.VMEM((XH,), jnp.int32),
        pltpu.VMEM((XH,), jnp.int32),
        pltpu.VMEM((XC,), jnp.float32),
        pltpu.VMEM((XC,), jnp.float32),
        pltpu.SemaphoreType.DMA,
        pltpu.SemaphoreType.DMA,
        pltpu.SemaphoreType.DMA,
        pltpu.SemaphoreType.DMA,
        pltpu.SemaphoreType.DMA,
    ],
)(_sc_body)


def kernel(x, W):
    xt = x.T
    wt = jnp.transpose(W, (0, 2, 1))
    ot = _emb(xt, wt)
    return jnp.transpose(ot, (2, 0, 1))


# speedup vs baseline: 1.1028x; 1.1028x over previous
import functools

import jax
import jax.numpy as jnp
from jax import lax
from jax.experimental import pallas as pl
from jax.experimental.pallas import tpu as pltpu
from jax.experimental.pallas import tpu_sc as plsc

F = 26
CARD = 100001
D = 32
B = 16384
NC = 2
NS = 16
NW = NC * NS
XC = 4096
NXC = B // XC
L = 16


def _sc_body(xt, wt, ot, tbl, xv0, xv1, xv2, ov0, ov1,
             tsem, xs0, xs1, xs2, os0, os1):
    e = lax.axis_index("s") * NC + lax.axis_index("c")
    xv = [xv0, xv1, xv2]
    ov = [ov0, ov1]
    xsem = [xs0, xs1, xs2]
    osem = [os0, os1]

    def do_field(f, carry):
        tcp = pltpu.async_copy(wt.at[f, e], tbl, tsem)
        xcp = [None, None, None]
        ocp = [None, None]
        xcp[0] = pltpu.async_copy(xt.at[f, pl.ds(0, XC)], xv[0], xsem[0])
        xcp[1] = pltpu.async_copy(xt.at[f, pl.ds(XC, XC)], xv[1], xsem[1])
        tcp.wait()
        for c in range(NXC):
            s = c % 3
            o = c % 2
            if c + 2 < NXC:
                s2 = (c + 2) % 3
                xcp[s2] = pltpu.async_copy(
                    xt.at[f, pl.ds((c + 2) * XC, XC)], xv[s2], xsem[s2])
            xcp[s].wait()
            if c >= 2:
                ocp[o].wait()

            def grp(i, carry2):
                for u in range(8):
                    idx = xv[s][pl.ds((i * 8 + u) * L, L)]
                    ov[o][pl.ds((i * 8 + u) * L, L)] = plsc.load_gather(tbl, [idx])
                return carry2
            lax.fori_loop(0, XC // L // 8, grp, 0)

            ocp[o] = pltpu.async_copy(
                ov[o], ot.at[f, e, pl.ds(c * XC, XC)], osem[o])
        ocp[0].wait()
        ocp[1].wait()
        return carry

    lax.fori_loop(0, F, do_field, 0)


_emb = functools.partial(
    pl.kernel,
    mesh=plsc.VectorSubcoreMesh(core_axis_name="c", subcore_axis_name="s"),
    out_type=jax.ShapeDtypeStruct((F, D, B), jnp.float32),
    compiler_params=pltpu.CompilerParams(needs_layout_passes=False),
    scratch_types=[
        pltpu.VMEM((CARD,), jnp.float32),
        pltpu.VMEM((XC,), jnp.int32),
        pltpu.VMEM((XC,), jnp.int32),
        pltpu.VMEM((XC,), jnp.int32),
        pltpu.VMEM((XC,), jnp.float32),
        pltpu.VMEM((XC,), jnp.float32),
        pltpu.SemaphoreType.DMA,
        pltpu.SemaphoreType.DMA,
        pltpu.SemaphoreType.DMA,
        pltpu.SemaphoreType.DMA,
        pltpu.SemaphoreType.DMA,
        pltpu.SemaphoreType.DMA,
    ],
)(_sc_body)


def kernel(x, W):
    xt = x.T
    wt = jnp.transpose(W, (0, 2, 1))
    ot = _emb(xt, wt)
    return jnp.transpose(ot, (2, 0, 1))


# speedup vs baseline: 1.1399x; 1.0336x over previous
import functools

import jax
import jax.numpy as jnp
from jax import lax
from jax.experimental import pallas as pl
from jax.experimental.pallas import tpu as pltpu
from jax.experimental.pallas import tpu_sc as plsc

F = 26
CARD = 100001
D = 32
B = 16384
NC = 2
NS = 16
NW = NC * NS
XC = 4096
NXC = B // XC
L = 16


def _sc_body(xt, wt, ot, tbl, xv0, xv1, ov0, ov1,
             tsem, xs0, xs1, os0, os1):
    e = lax.axis_index("s") * NC + lax.axis_index("c")
    xv = [xv0, xv1]
    ov = [ov0, ov1]
    xsem = [xs0, xs1]
    osem = [os0, os1]

    def do_field(f, carry):
        tcp = pltpu.async_copy(wt.at[f, e], tbl, tsem)
        xcp = [None, None]
        ocp = [None, None]
        xcp[0] = pltpu.async_copy(xt.at[f, pl.ds(0, XC)], xv[0], xsem[0])
        tcp.wait()
        for c in range(NXC):
            s = c % 2
            if c + 1 < NXC:
                xcp[s ^ 1] = pltpu.async_copy(
                    xt.at[f, pl.ds((c + 1) * XC, XC)], xv[s ^ 1], xsem[s ^ 1])
            xcp[s].wait()
            if c >= 2:
                ocp[s].wait()

            def grp(i, carry2):
                for u in range(8):
                    idx = xv[s][pl.ds((i * 8 + u) * L, L)]
                    ov[s][pl.ds((i * 8 + u) * L, L)] = plsc.load_gather(tbl, [idx])
                return carry2
            lax.fori_loop(0, XC // L // 8, grp, 0)

            ocp[s] = pltpu.async_copy(
                ov[s], ot.at[f, e, pl.ds(c * XC, XC)], osem[s])
        ocp[0].wait()
        ocp[1].wait()
        return carry

    lax.fori_loop(0, F, do_field, 0)


_emb = functools.partial(
    pl.kernel,
    mesh=plsc.VectorSubcoreMesh(core_axis_name="c", subcore_axis_name="s"),
    out_type=jax.ShapeDtypeStruct((F, D, B), jnp.float32),
    compiler_params=pltpu.CompilerParams(needs_layout_passes=False),
    scratch_types=[
        pltpu.VMEM((CARD,), jnp.float32),
        pltpu.VMEM((XC,), jnp.int32),
        pltpu.VMEM((XC,), jnp.int32),
        pltpu.VMEM((XC,), jnp.float32),
        pltpu.VMEM((XC,), jnp.float32),
        pltpu.SemaphoreType.DMA,
        pltpu.SemaphoreType.DMA,
        pltpu.SemaphoreType.DMA,
        pltpu.SemaphoreType.DMA,
        pltpu.SemaphoreType.DMA,
    ],
)(_sc_body)


def kernel(x, W):
    xt = x.T
    wt = jnp.transpose(W, (0, 2, 1))
    ot = _emb(xt, wt)
    return jnp.transpose(ot, (2, 0, 1))


# speedup vs baseline: 1.1410x; 1.0010x over previous
import functools

import jax
import jax.numpy as jnp
from jax import lax
from jax.experimental import pallas as pl
from jax.experimental.pallas import tpu as pltpu
from jax.experimental.pallas import tpu_sc as plsc

F = 26
CARD = 100001
D = 32
B = 16384
NC = 2
NS = 16
NW = NC * NS
XC = 4096
NXC = B // XC
L = 16


def _sc_body(xt, wt, ot, tbl, xv0, xv1, ov0, ov1,
             tsem, xs0, xs1, os0, os1):
    e = lax.axis_index("c") * NS + lax.axis_index("s")
    xv = [xv0, xv1]
    ov = [ov0, ov1]
    xsem = [xs0, xs1]
    osem = [os0, os1]

    def do_field(f, carry):
        tcp = pltpu.async_copy(wt.at[f, e], tbl, tsem)
        xcp = [None, None]
        ocp = [None, None]
        xcp[0] = pltpu.async_copy(xt.at[f, pl.ds(0, XC)], xv[0], xsem[0])
        tcp.wait()
        for c in range(NXC):
            s = c % 2
            if c + 1 < NXC:
                xcp[s ^ 1] = pltpu.async_copy(
                    xt.at[f, pl.ds((c + 1) * XC, XC)], xv[s ^ 1], xsem[s ^ 1])
            xcp[s].wait()
            if c >= 2:
                ocp[s].wait()

            def grp(i, carry2):
                for u in range(8):
                    idx = xv[s][pl.ds((i * 8 + u) * L, L)]
                    ov[s][pl.ds((i * 8 + u) * L, L)] = plsc.load_gather(tbl, [idx])
                return carry2
            lax.fori_loop(0, XC // L // 8, grp, 0)

            ocp[s] = pltpu.async_copy(
                ov[s], ot.at[f, e, pl.ds(c * XC, XC)], osem[s])
        ocp[0].wait()
        ocp[1].wait()
        return carry

    lax.fori_loop(0, F, do_field, 0)


_emb = functools.partial(
    pl.kernel,
    mesh=plsc.VectorSubcoreMesh(core_axis_name="c", subcore_axis_name="s"),
    out_type=jax.ShapeDtypeStruct((F, D, B), jnp.float32),
    compiler_params=pltpu.CompilerParams(needs_layout_passes=False),
    scratch_types=[
        pltpu.VMEM((CARD,), jnp.float32),
        pltpu.VMEM((XC,), jnp.int32),
        pltpu.VMEM((XC,), jnp.int32),
        pltpu.VMEM((XC,), jnp.float32),
        pltpu.VMEM((XC,), jnp.float32),
        pltpu.SemaphoreType.DMA,
        pltpu.SemaphoreType.DMA,
        pltpu.SemaphoreType.DMA,
        pltpu.SemaphoreType.DMA,
        pltpu.SemaphoreType.DMA,
    ],
)(_sc_body)


def kernel(x, W):
    xt = x.T
    wt = jnp.transpose(W, (0, 2, 1))
    ot = _emb(xt, wt)
    return jnp.transpose(ot, (2, 0, 1))


# speedup vs baseline: 1.1462x; 1.0046x over previous
import functools

import jax
import jax.numpy as jnp
from jax import lax
from jax.experimental import pallas as pl
from jax.experimental.pallas import tpu as pltpu
from jax.experimental.pallas import tpu_sc as plsc

F = 26
CARD = 100001
D = 32
B = 16384
NC = 2
NS = 16
NW = NC * NS
XC = 4096
NXC = B // XC
L = 16


def _sc_body(xt, wt, ot, tbl, xv0, xv1, ov0, ov1,
             tsem, xs0, xs1, os0, os1):
    e = lax.axis_index("c") * NS + lax.axis_index("s")
    xv = [xv0, xv1]
    ov = [ov0, ov1]
    xsem = [xs0, xs1]
    osem = [os0, os1]

    def do_field(f, carry):
        tcp = pltpu.async_copy(wt.at[f, e], tbl, tsem)
        xcp = [None, None]
        ocp = [None, None]
        xcp[0] = pltpu.async_copy(xt.at[f, pl.ds(0, XC)], xv[0], xsem[0])
        tcp.wait()
        for c in range(NXC):
            s = c % 2
            if c + 1 < NXC:
                xcp[s ^ 1] = pltpu.async_copy(
                    xt.at[f, pl.ds((c + 1) * XC, XC)], xv[s ^ 1], xsem[s ^ 1])
            xcp[s].wait()
            if c >= 2:
                ocp[s].wait()

            def grp(i, carry2):
                for u in range(16):
                    idx = xv[s][pl.ds((i * 16 + u) * L, L)]
                    ov[s][pl.ds((i * 16 + u) * L, L)] = plsc.load_gather(tbl, [idx])
                return carry2
            lax.fori_loop(0, XC // L // 16, grp, 0)

            ocp[s] = pltpu.async_copy(
                ov[s], ot.at[f, e, pl.ds(c * XC, XC)], osem[s])
        ocp[0].wait()
        ocp[1].wait()
        return carry

    lax.fori_loop(0, F, do_field, 0)


_emb = functools.partial(
    pl.kernel,
    mesh=plsc.VectorSubcoreMesh(core_axis_name="c", subcore_axis_name="s"),
    out_type=jax.ShapeDtypeStruct((F, D, B), jnp.float32),
    compiler_params=pltpu.CompilerParams(needs_layout_passes=False),
    scratch_types=[
        pltpu.VMEM((CARD,), jnp.float32),
        pltpu.VMEM((XC,), jnp.int32),
        pltpu.VMEM((XC,), jnp.int32),
        pltpu.VMEM((XC,), jnp.float32),
        pltpu.VMEM((XC,), jnp.float32),
        pltpu.SemaphoreType.DMA,
        pltpu.SemaphoreType.DMA,
        pltpu.SemaphoreType.DMA,
        pltpu.SemaphoreType.DMA,
        pltpu.SemaphoreType.DMA,
    ],
)(_sc_body)


def kernel(x, W):
    xt = x.T
    wt = jnp.transpose(W, (0, 2, 1))
    ot = _emb(xt, wt)
    return jnp.transpose(ot, (2, 0, 1))
